# baseline (speedup 1.0000x reference)
import functools

import jax
import jax.numpy as jnp
from jax.experimental import pallas as pl
from jax.experimental.pallas import tpu as pltpu

N_EMBD = 2048
D_FF = 8192
NUM_EXPERTS = 8
TOP_K = 2
T_TOKENS = 2048

TM = 512
TF = 1024
GRID_M = T_TOKENS // TM
GRID_F = D_FF // TF


def _router_body(x_ref, wg_ref, bg_ref, w_ref):
    s = jnp.dot(x_ref[...], wg_ref[...], preferred_element_type=jnp.float32)
    s = s + bg_ref[...]
    lane = jax.lax.broadcasted_iota(jnp.int32, s.shape, 1)
    m1 = jnp.max(s, axis=1, keepdims=True)
    e1 = jnp.min(jnp.where(s == m1, lane, NUM_EXPERTS), axis=1, keepdims=True)
    s2 = jnp.where(lane == e1, -jnp.inf, s)
    m2 = jnp.max(s2, axis=1, keepdims=True)
    e2 = jnp.min(jnp.where(s2 == m2, lane, NUM_EXPERTS), axis=1, keepdims=True)
    g2 = 1.0 / (1.0 + jnp.exp(m1 - m2))
    g1 = 1.0 - g2
    w_ref[...] = jnp.where(lane == e1, g1, 0.0) + jnp.where(lane == e2, g2, 0.0)


def _moe_body(w_ref, x_ref, wfc_ref, bfc_ref, wproj_ref, bproj_ref, out_ref):
    e = pl.program_id(0)
    f = pl.program_id(1)
    m = pl.program_id(2)

    xb = x_ref[...]
    h = jnp.dot(xb, wfc_ref[0], preferred_element_type=jnp.float32)
    h = h + bfc_ref[0]
    h = 0.5 * h * (1.0 + jnp.tanh(0.7978845608028654 * (h + 0.044715 * h * h * h)))
    yb = jnp.dot(h.astype(jnp.bfloat16), wproj_ref[0],
                 preferred_element_type=jnp.float32)
    yb = jnp.where(f == 0, yb + bproj_ref[0], yb)

    wblk = w_ref[pl.ds(m * TM, TM), :]
    lane = jax.lax.broadcasted_iota(jnp.int32, wblk.shape, 1)
    wcol = jnp.sum(jnp.where(lane == e, wblk, 0.0), axis=1, keepdims=True)
    contrib = wcol * yb

    @pl.when(jnp.logical_and(e == 0, f == 0))
    def _init():
        out_ref[pl.ds(m * TM, TM), :] = contrib

    @pl.when(jnp.logical_or(e != 0, f != 0))
    def _acc():
        out_ref[pl.ds(m * TM, TM), :] = out_ref[pl.ds(m * TM, TM), :] + contrib


def kernel(x, Wg, bg, Wfc, bfc, Wproj, bproj):
    Bb, Tt, C = x.shape
    x_flat = x.reshape(Tt, C)

    w = pl.pallas_call(
        _router_body,
        out_shape=jax.ShapeDtypeStruct((Tt, NUM_EXPERTS), jnp.float32),
        in_specs=[
            pl.BlockSpec((Tt, C), lambda: (0, 0)),
            pl.BlockSpec((C, NUM_EXPERTS), lambda: (0, 0)),
            pl.BlockSpec((1, NUM_EXPERTS), lambda: (0, 0)),
        ],
        out_specs=pl.BlockSpec((Tt, NUM_EXPERTS), lambda: (0, 0)),
    )(x_flat, Wg, bg.reshape(1, NUM_EXPERTS))

    x_bf = x_flat.astype(jnp.bfloat16)
    wfc_bf = Wfc.astype(jnp.bfloat16)
    wproj_bf = Wproj.astype(jnp.bfloat16)

    out = pl.pallas_call(
        _moe_body,
        grid=(NUM_EXPERTS, GRID_F, GRID_M),
        out_shape=jax.ShapeDtypeStruct((Tt, C), jnp.float32),
        in_specs=[
            pl.BlockSpec((Tt, NUM_EXPERTS), lambda e, f, m: (0, 0)),
            pl.BlockSpec((TM, C), lambda e, f, m: (m, 0)),
            pl.BlockSpec((1, C, TF), lambda e, f, m: (e, 0, f)),
            pl.BlockSpec((1, 1, TF), lambda e, f, m: (e, 0, f)),
            pl.BlockSpec((1, TF, C), lambda e, f, m: (e, f, 0)),
            pl.BlockSpec((1, 1, C), lambda e, f, m: (e, 0, 0)),
        ],
        out_specs=pl.BlockSpec((Tt, C), lambda e, f, m: (0, 0)),
    )(w, x_bf, wfc_bf, bfc.reshape(NUM_EXPERTS, 1, D_FF),
      wproj_bf, bproj.reshape(NUM_EXPERTS, 1, C))

    return out.reshape(Bb, Tt, C)


# speedup vs baseline: 1.0866x; 1.0866x over previous
import functools

import jax
import jax.numpy as jnp
from jax.experimental import pallas as pl
from jax.experimental.pallas import tpu as pltpu

N_EMBD = 2048
D_FF = 8192
NUM_EXPERTS = 8
TOP_K = 2
T_TOKENS = 2048

TM = 512
TF = 512
GRID_M = T_TOKENS // TM
GRID_F = D_FF // TF


def _router_body(x_ref, wg_ref, bg_ref, w_ref):
    s = jnp.dot(x_ref[...], wg_ref[...], preferred_element_type=jnp.float32)
    s = s + bg_ref[...]
    lane = jax.lax.broadcasted_iota(jnp.int32, s.shape, 1)
    m1 = jnp.max(s, axis=1, keepdims=True)
    e1 = jnp.min(jnp.where(s == m1, lane, NUM_EXPERTS), axis=1, keepdims=True)
    s2 = jnp.where(lane == e1, -jnp.inf, s)
    m2 = jnp.max(s2, axis=1, keepdims=True)
    e2 = jnp.min(jnp.where(s2 == m2, lane, NUM_EXPERTS), axis=1, keepdims=True)
    g2 = 1.0 / (1.0 + jnp.exp(m1 - m2))
    g1 = 1.0 - g2
    w_ref[...] = jnp.where(lane == e1, g1, 0.0) + jnp.where(lane == e2, g2, 0.0)


def _moe_body(w_ref, x_ref, wfc_ref, bfc_ref, wproj_ref, bproj_ref, out_ref):
    e = pl.program_id(0)
    f = pl.program_id(1)
    m = pl.program_id(2)

    xb = x_ref[...]
    h = jnp.dot(xb, wfc_ref[0], preferred_element_type=jnp.float32)
    h = h + bfc_ref[0]
    h = 0.5 * h * (1.0 + jnp.tanh(0.7978845608028654 * (h + 0.044715 * h * h * h)))
    yb = jnp.dot(h, wproj_ref[0], preferred_element_type=jnp.float32)
    yb = jnp.where(f == 0, yb + bproj_ref[0], yb)

    wblk = w_ref[pl.ds(m * TM, TM), :]
    lane = jax.lax.broadcasted_iota(jnp.int32, wblk.shape, 1)
    wcol = jnp.sum(jnp.where(lane == e, wblk, 0.0), axis=1, keepdims=True)
    contrib = wcol * yb

    @pl.when(jnp.logical_and(e == 0, f == 0))
    def _init():
        out_ref[pl.ds(m * TM, TM), :] = contrib

    @pl.when(jnp.logical_or(e != 0, f != 0))
    def _acc():
        out_ref[pl.ds(m * TM, TM), :] = out_ref[pl.ds(m * TM, TM), :] + contrib


def kernel(x, Wg, bg, Wfc, bfc, Wproj, bproj):
    Bb, Tt, C = x.shape
    x_flat = x.reshape(Tt, C)

    w = pl.pallas_call(
        _router_body,
        out_shape=jax.ShapeDtypeStruct((Tt, NUM_EXPERTS), jnp.float32),
        in_specs=[
            pl.BlockSpec((Tt, C), lambda: (0, 0)),
            pl.BlockSpec((C, NUM_EXPERTS), lambda: (0, 0)),
            pl.BlockSpec((1, NUM_EXPERTS), lambda: (0, 0)),
        ],
        out_specs=pl.BlockSpec((Tt, NUM_EXPERTS), lambda: (0, 0)),
    )(x_flat, Wg, bg.reshape(1, NUM_EXPERTS))

    out = pl.pallas_call(
        _moe_body,
        grid=(NUM_EXPERTS, GRID_F, GRID_M),
        out_shape=jax.ShapeDtypeStruct((Tt, C), jnp.float32),
        in_specs=[
            pl.BlockSpec((Tt, NUM_EXPERTS), lambda e, f, m: (0, 0)),
            pl.BlockSpec((TM, C), lambda e, f, m: (m, 0)),
            pl.BlockSpec((1, C, TF), lambda e, f, m: (e, 0, f)),
            pl.BlockSpec((1, 1, TF), lambda e, f, m: (e, 0, f)),
            pl.BlockSpec((1, TF, C), lambda e, f, m: (e, f, 0)),
            pl.BlockSpec((1, 1, C), lambda e, f, m: (e, 0, 0)),
        ],
        out_specs=pl.BlockSpec((Tt, C), lambda e, f, m: (0, 0)),
    )(w, x_flat, Wfc, bfc.reshape(NUM_EXPERTS, 1, D_FF),
      Wproj, bproj.reshape(NUM_EXPERTS, 1, C))

    return out.reshape(Bb, Tt, C)


# speedup vs baseline: 1.2022x; 1.1063x over previous
import functools

import jax
import jax.numpy as jnp
from jax.experimental import pallas as pl
from jax.experimental.pallas import tpu as pltpu

N_EMBD = 2048
D_FF = 8192
NUM_EXPERTS = 8
TOP_K = 2
T_TOKENS = 2048

TM = 512
TF = 512
GRID_M = T_TOKENS // TM
GRID_F = D_FF // TF


def _router_body(x_ref, wg_ref, bg_ref, w_ref):
    s = jnp.dot(x_ref[...], wg_ref[...], preferred_element_type=jnp.float32)
    s = s + bg_ref[...]
    lane = jax.lax.broadcasted_iota(jnp.int32, s.shape, 1)
    m1 = jnp.max(s, axis=1, keepdims=True)
    e1 = jnp.min(jnp.where(s == m1, lane, NUM_EXPERTS), axis=1, keepdims=True)
    s2 = jnp.where(lane == e1, -jnp.inf, s)
    m2 = jnp.max(s2, axis=1, keepdims=True)
    e2 = jnp.min(jnp.where(s2 == m2, lane, NUM_EXPERTS), axis=1, keepdims=True)
    g2 = 1.0 / (1.0 + jnp.exp(m1 - m2))
    g1 = 1.0 - g2
    w_ref[...] = jnp.where(lane == e1, g1, 0.0) + jnp.where(lane == e2, g2, 0.0)


def _moe_body(w_ref, x_ref, wfc_ref, bfc_ref, wproj_ref, bproj_ref, out_ref):
    e = pl.program_id(0)
    f = pl.program_id(1)
    m = pl.program_id(2)

    xb = x_ref[pl.ds(m * TM, TM), :]
    h = jnp.dot(xb, wfc_ref[0], preferred_element_type=jnp.float32)
    h = h + bfc_ref[0]
    h = 0.5 * h * (1.0 + jnp.tanh(0.7978845608028654 * (h + 0.044715 * h * h * h)))
    yb = jnp.dot(h, wproj_ref[0], preferred_element_type=jnp.float32)
    yb = jnp.where(f == 0, yb + bproj_ref[0], yb)

    wblk = w_ref[pl.ds(m * TM, TM), :]
    lane = jax.lax.broadcasted_iota(jnp.int32, wblk.shape, 1)
    wcol = jnp.sum(jnp.where(lane == e, wblk, 0.0), axis=1, keepdims=True)
    contrib = wcol * yb

    @pl.when(jnp.logical_and(e == 0, f == 0))
    def _init():
        out_ref[pl.ds(m * TM, TM), :] = contrib

    @pl.when(jnp.logical_or(e != 0, f != 0))
    def _acc():
        out_ref[pl.ds(m * TM, TM), :] = out_ref[pl.ds(m * TM, TM), :] + contrib


def kernel(x, Wg, bg, Wfc, bfc, Wproj, bproj):
    Bb, Tt, C = x.shape
    x_flat = x.reshape(Tt, C)

    w = pl.pallas_call(
        _router_body,
        out_shape=jax.ShapeDtypeStruct((Tt, NUM_EXPERTS), jnp.float32),
        in_specs=[
            pl.BlockSpec((Tt, C), lambda: (0, 0)),
            pl.BlockSpec((C, NUM_EXPERTS), lambda: (0, 0)),
            pl.BlockSpec((1, NUM_EXPERTS), lambda: (0, 0)),
        ],
        out_specs=pl.BlockSpec((Tt, NUM_EXPERTS), lambda: (0, 0)),
    )(x_flat, Wg, bg.reshape(1, NUM_EXPERTS))

    out = pl.pallas_call(
        _moe_body,
        grid=(NUM_EXPERTS, GRID_F, GRID_M),
        out_shape=jax.ShapeDtypeStruct((Tt, C), jnp.float32),
        in_specs=[
            pl.BlockSpec((Tt, NUM_EXPERTS), lambda e, f, m: (0, 0)),
            pl.BlockSpec((Tt, C), lambda e, f, m: (0, 0)),
            pl.BlockSpec((1, C, TF), lambda e, f, m: (e, 0, f)),
            pl.BlockSpec((1, 1, TF), lambda e, f, m: (e, 0, f)),
            pl.BlockSpec((1, TF, C), lambda e, f, m: (e, f, 0)),
            pl.BlockSpec((1, 1, C), lambda e, f, m: (e, 0, 0)),
        ],
        out_specs=pl.BlockSpec((Tt, C), lambda e, f, m: (0, 0)),
    )(w, x_flat, Wfc, bfc.reshape(NUM_EXPERTS, 1, D_FF),
      Wproj, bproj.reshape(NUM_EXPERTS, 1, C))

    return out.reshape(Bb, Tt, C)


# speedup vs baseline: 2.5021x; 2.0814x over previous
import dataclasses
import functools

import jax
import jax.numpy as jnp
from jax import lax
from jax.experimental import pallas as pl
from jax.experimental.pallas import tpu as pltpu
from jax.experimental.pallas import tpu_sc as plsc

T = 2048
C = 2048
DFF = 8192
E = 8
TM = 512
TF = 512
GRID_F = DFF // TF
MAX_TILES = 16
ROWS = MAX_TILES * TM
NW = 16
PAIRS = 2 * T
PPW = PAIRS // NW
VPW = PPW // 16


def _router_body(x_ref, wg_ref, bg_ref, eids_ref, gates_ref):
    s = jnp.dot(x_ref[...], wg_ref[...], preferred_element_type=jnp.float32)
    s = s + bg_ref[...]
    lane = lax.broadcasted_iota(jnp.int32, s.shape, 1)
    m1 = jnp.max(s, axis=1, keepdims=True)
    e1 = jnp.min(jnp.where(s == m1, lane, E), axis=1, keepdims=True)
    s2 = jnp.where(lane == e1, -jnp.inf, s)
    m2 = jnp.max(s2, axis=1, keepdims=True)
    e2 = jnp.min(jnp.where(s2 == m2, lane, E), axis=1, keepdims=True)
    g2 = 1.0 / (1.0 + jnp.exp(m1 - m2))
    g1 = 1.0 - g2
    eids_ref[...] = jnp.concatenate([e1, e2], axis=1).astype(jnp.int32)
    gates_ref[...] = jnp.concatenate([g1, g2], axis=1)


def _router(x_flat, Wg, bg):
    return pl.pallas_call(
        _router_body,
        out_shape=(
            jax.ShapeDtypeStruct((T, 2), jnp.int32),
            jax.ShapeDtypeStruct((T, 2), jnp.float32),
        ),
        in_specs=[
            pl.BlockSpec((T, C), lambda: (0, 0)),
            pl.BlockSpec((C, E), lambda: (0, 0)),
            pl.BlockSpec((1, E), lambda: (0, 0)),
        ],
        out_specs=(
            pl.BlockSpec((T, 2), lambda: (0, 0)),
            pl.BlockSpec((T, 2), lambda: (0, 0)),
        ),
    )(x_flat, Wg, bg.reshape(1, E))


def _sortmeta_body(eids_hbm, pos0_hbm, pos1_hbm, meta_hbm,
                   eidv, rankv_m, histv, allh, shh, startv, outv):
    cid = lax.axis_index("c")
    wid = lax.axis_index("s")
    iota = lax.iota(jnp.int32, 16)

    @pl.when(cid == 0)
    def _work():
        pltpu.sync_copy(eids_hbm.at[pl.ds(wid * PPW, PPW)], eidv)

        def vbody(v, cnts):
            ev = eidv[pl.ds(v * 16, 16)]
            rk = jnp.zeros((16,), jnp.int32)
            new = []
            for e in range(E):
                m = ev == e
                mi = m.astype(jnp.int32)
                pref = plsc.cumsum(mi)
                tot = jnp.sum(mi)
                rk = jnp.where(m, cnts[e] + pref - 1, rk)
                new.append(cnts[e] + tot)
            rankv_m[pl.ds(v * 16, 16)] = rk
            return tuple(new)

        cnts = lax.fori_loop(0, VPW, vbody,
                             tuple(jnp.int32(0) for _ in range(E)))

        hv = jnp.zeros((16,), jnp.int32)
        for e in range(E):
            hv = jnp.where(iota == e, cnts[e], hv)
        histv[...] = hv
        pltpu.sync_copy(histv, shh.at[wid])
        plsc.subcore_barrier()

        pltpu.sync_copy(shh, allh)
        base = jnp.zeros((16,), jnp.int32)
        tot = jnp.zeros((16,), jnp.int32)
        for j in range(NW):
            hj = allh[j, :]
            base = base + jnp.where(j < wid, hj, 0)
            tot = tot + hj
        padded = ((tot + (TM - 1)) >> 9) << 9
        o_incl = plsc.cumsum(padded)
        o_excl = o_incl - padded
        startv[...] = o_excl + base

        def pbody(v, _):
            ev = eidv[pl.ds(v * 16, 16)]
            st = plsc.load_gather(startv, [ev])
            rankv_m[pl.ds(v * 16, 16)] = st + rankv_m[pl.ds(v * 16, 16)]
            return 0

        lax.fori_loop(0, VPW, pbody, 0)

        half = PPW // 2
        for u in range(VPW // 2):
            idx0 = u * 32 + 2 * iota
            outv[pl.ds(u * 16, 16)] = plsc.load_gather(rankv_m, [idx0])
            outv[pl.ds(half + u * 16, 16)] = plsc.load_gather(rankv_m, [idx0 + 1])
        pltpu.sync_copy(outv.at[pl.ds(0, half)],
                        pos0_hbm.at[pl.ds(wid * half, half)])
        pltpu.sync_copy(outv.at[pl.ds(half, half)],
                        pos1_hbm.at[pl.ds(wid * half, half)])

        @pl.when(wid == 0)
        def _meta():
            tev = jnp.zeros((16,), jnp.int32)
            for e in range(E):
                oe = jnp.sum(jnp.where(iota == e, o_incl, 0))
                tev = tev + (oe <= iota * TM).astype(jnp.int32)
            last_e = jnp.max(jnp.where(tot > 0, iota, 0))
            tev = jnp.minimum(tev, last_e)
            nt = jnp.sum(padded) >> 9
            histv[...] = tev
            pltpu.sync_copy(histv, meta_hbm.at[pl.ds(0, 16)])
            histv[...] = jnp.zeros((16,), jnp.int32) + nt
            pltpu.sync_copy(histv, meta_hbm.at[pl.ds(16, 16)])


def _sc_compiler_params():
    cp = pltpu.CompilerParams()
    if "needs_layout_passes" in pltpu.CompilerParams.__dataclass_fields__:
        cp = dataclasses.replace(cp, needs_layout_passes=False)
    return cp


def _sortmeta(eids_flat):
    mesh = plsc.VectorSubcoreMesh(core_axis_name="c", subcore_axis_name="s")
    kern = functools.partial(
        pl.kernel,
        compiler_params=_sc_compiler_params(),
        out_type=(
            jax.ShapeDtypeStruct((T,), jnp.int32),
            jax.ShapeDtypeStruct((T,), jnp.int32),
            jax.ShapeDtypeStruct((32,), jnp.int32),
        ),
        mesh=mesh,
        scratch_types=[
            pltpu.VMEM((PPW,), jnp.int32),
            pltpu.VMEM((PPW,), jnp.int32),
            pltpu.VMEM((16,), jnp.int32),
            pltpu.VMEM((NW, 16), jnp.int32),
            pltpu.VMEM_SHARED((NW, 16), jnp.int32),
            pltpu.VMEM((16,), jnp.int32),
            pltpu.VMEM((PPW,), jnp.int32),
        ],
    )(_sortmeta_body)
    return kern(eids_flat)


def _dispatch_body(x_hbm, pos0_hbm, pos1_hbm, xs_hbm, rows, i0, i1, sem):
    cid = lax.axis_index("c")
    sid = lax.axis_index("s")
    w = sid * 2 + cid
    tpw = T // 32
    for chunk in range(2):
        base = w * tpw + chunk * 32
        pltpu.sync_copy(x_hbm.at[pl.ds(base, 32)], rows)
        pltpu.sync_copy(pos0_hbm.at[pl.ds(base, 32)], i0)
        pltpu.sync_copy(pos1_hbm.at[pl.ds(base, 32)], i1)
        pltpu.async_copy(rows, xs_hbm.at[i0], sem).wait()
        pltpu.async_copy(rows, xs_hbm.at[i1], sem).wait()


def _dispatch(x_flat, pos0, pos1):
    mesh = plsc.VectorSubcoreMesh(core_axis_name="c", subcore_axis_name="s")
    kern = functools.partial(
        pl.kernel,
        out_type=jax.ShapeDtypeStruct((ROWS, C), jnp.float32),
        mesh=mesh,
        scratch_types=[
            pltpu.VMEM((32, C), jnp.float32),
            pltpu.VMEM((32,), jnp.int32),
            pltpu.VMEM((32,), jnp.int32),
            pltpu.SemaphoreType.DMA,
        ],
    )(_dispatch_body)
    return kern(x_flat, pos0, pos1)


def _ffn_body(meta_ref, xs_ref, wfc_ref, bfc_ref, wproj_ref, bproj_ref, o_ref):
    j = pl.program_id(0)
    f = pl.program_id(1)
    nt = meta_ref[16]

    @pl.when(j < nt)
    def _active():
        h = jnp.dot(xs_ref[...], wfc_ref[0], preferred_element_type=jnp.float32)
        h = h + bfc_ref[0]
        h = 0.5 * h * (1.0 + jnp.tanh(0.7978845608028654
                                      * (h + 0.044715 * h * h * h)))
        yb = jnp.dot(h, wproj_ref[0], preferred_element_type=jnp.float32)

        @pl.when(f == 0)
        def _():
            o_ref[...] = yb + bproj_ref[0]

        @pl.when(f != 0)
        def _():
            o_ref[...] = o_ref[...] + yb


def _ffn(meta, xs, Wfc, bfc, Wproj, bproj):
    def jpin(j, f, m):
        return jnp.minimum(j, m[16] - 1)

    def fpin(j, f, m):
        return jnp.where(j < m[16], f, GRID_F - 1)

    grid_spec = pltpu.PrefetchScalarGridSpec(
        num_scalar_prefetch=1,
        grid=(MAX_TILES, GRID_F),
        in_specs=[
            pl.BlockSpec((TM, C), lambda j, f, m: (jpin(j, f, m), 0)),
            pl.BlockSpec((1, C, TF), lambda j, f, m: (m[j], 0, fpin(j, f, m))),
            pl.BlockSpec((1, 1, TF), lambda j, f, m: (m[j], 0, fpin(j, f, m))),
            pl.BlockSpec((1, TF, C), lambda j, f, m: (m[j], fpin(j, f, m), 0)),
            pl.BlockSpec((1, 1, C), lambda j, f, m: (m[j], 0, 0)),
        ],
        out_specs=pl.BlockSpec((TM, C), lambda j, f, m: (jpin(j, f, m), 0)),
    )
    return pl.pallas_call(
        _ffn_body,
        grid_spec=grid_spec,
        out_shape=jax.ShapeDtypeStruct((ROWS, C), jnp.float32),
    )(meta, xs, Wfc, bfc.reshape(E, 1, DFF), Wproj, bproj.reshape(E, 1, C))


def _gather_body(y_hbm, pos0_hbm, pos1_hbm, yc0_hbm, yc1_hbm, rows, idx, sem):
    cid = lax.axis_index("c")
    sid = lax.axis_index("s")
    w = sid * 2 + cid
    tpw = T // 32
    for chunk in range(2):
        base = w * tpw + chunk * 32
        pltpu.sync_copy(pos0_hbm.at[pl.ds(base, 32)], idx)
        pltpu.async_copy(y_hbm.at[idx], rows, sem).wait()
        pltpu.sync_copy(rows, yc0_hbm.at[pl.ds(base, 32)])
        pltpu.sync_copy(pos1_hbm.at[pl.ds(base, 32)], idx)
        pltpu.async_copy(y_hbm.at[idx], rows, sem).wait()
        pltpu.sync_copy(rows, yc1_hbm.at[pl.ds(base, 32)])


def _gather(y, pos0, pos1):
    mesh = plsc.VectorSubcoreMesh(core_axis_name="c", subcore_axis_name="s")
    kern = functools.partial(
        pl.kernel,
        out_type=(
            jax.ShapeDtypeStruct((T, C), jnp.float32),
            jax.ShapeDtypeStruct((T, C), jnp.float32),
        ),
        mesh=mesh,
        scratch_types=[
            pltpu.VMEM((32, C), jnp.float32),
            pltpu.VMEM((32,), jnp.int32),
            pltpu.SemaphoreType.DMA,
        ],
    )(_gather_body)
    return kern(y, pos0, pos1)


def _combine_body(y0_ref, y1_ref, g_ref, o_ref):
    g = g_ref[...]
    o_ref[...] = y0_ref[...] * g[:, 0:1] + y1_ref[...] * g[:, 1:2]


def _combine(yc0, yc1, gates):
    grid_m = T // TM
    return pl.pallas_call(
        _combine_body,
        grid=(grid_m,),
        out_shape=jax.ShapeDtypeStruct((T, C), jnp.float32),
        in_specs=[
            pl.BlockSpec((TM, C), lambda m: (m, 0)),
            pl.BlockSpec((TM, C), lambda m: (m, 0)),
            pl.BlockSpec((TM, 2), lambda m: (m, 0)),
        ],
        out_specs=pl.BlockSpec((TM, C), lambda m: (m, 0)),
    )(yc0, yc1, gates)


def kernel(x, Wg, bg, Wfc, bfc, Wproj, bproj):
    Bb, Tt, Cc = x.shape
    x_flat = x.reshape(Tt, Cc)

    eids, gates = _router(x_flat, Wg, bg)
    pos0, pos1, meta = _sortmeta(eids.reshape(PAIRS))
    xs = _dispatch(x_flat, pos0, pos1)
    y = _ffn(meta, xs, Wfc, bfc, Wproj, bproj)
    yc0, yc1 = _gather(y, pos0, pos1)
    out = _combine(yc0, yc1, gates)
    return out.reshape(Bb, Tt, Cc)


# speedup vs baseline: 2.7962x; 1.1175x over previous
import dataclasses
import functools

import jax
import jax.numpy as jnp
from jax import lax
from jax.experimental import pallas as pl
from jax.experimental.pallas import tpu as pltpu
from jax.experimental.pallas import tpu_sc as plsc

T = 2048
C = 2048
DFF = 8192
E = 8
TM = 512
TF = 1024
GRID_F = DFF // TF
MAX_TILES = 16
ROWS = MAX_TILES * TM
NW = 16
PAIRS = 2 * T
PPW = PAIRS // NW
VPW = PPW // 16


def _router_body(x_ref, wg_ref, bg_ref, eids_ref, gates_ref):
    s = jnp.dot(x_ref[...], wg_ref[...], preferred_element_type=jnp.float32)
    s = s + bg_ref[...]
    lane = lax.broadcasted_iota(jnp.int32, s.shape, 1)
    m1 = jnp.max(s, axis=1, keepdims=True)
    e1 = jnp.min(jnp.where(s == m1, lane, E), axis=1, keepdims=True)
    s2 = jnp.where(lane == e1, -jnp.inf, s)
    m2 = jnp.max(s2, axis=1, keepdims=True)
    e2 = jnp.min(jnp.where(s2 == m2, lane, E), axis=1, keepdims=True)
    g2 = 1.0 / (1.0 + jnp.exp(m1 - m2))
    g1 = 1.0 - g2
    eids_ref[...] = jnp.concatenate([e1, e2], axis=1).astype(jnp.int32)
    gates_ref[...] = jnp.concatenate([g1, g2], axis=1)


def _router(x_flat, Wg, bg):
    return pl.pallas_call(
        _router_body,
        out_shape=(
            jax.ShapeDtypeStruct((T, 2), jnp.int32),
            jax.ShapeDtypeStruct((T, 2), jnp.float32),
        ),
        in_specs=[
            pl.BlockSpec((T, C), lambda: (0, 0)),
            pl.BlockSpec((C, E), lambda: (0, 0)),
            pl.BlockSpec((1, E), lambda: (0, 0)),
        ],
        out_specs=(
            pl.BlockSpec((T, 2), lambda: (0, 0)),
            pl.BlockSpec((T, 2), lambda: (0, 0)),
        ),
    )(x_flat, Wg, bg.reshape(1, E))


def _sortmeta_body(eids_hbm, pos0_hbm, pos1_hbm, meta_hbm,
                   eidv, rankv_m, histv, allh, shh, startv, outv):
    cid = lax.axis_index("c")
    wid = lax.axis_index("s")
    iota = lax.iota(jnp.int32, 16)

    @pl.when(cid == 0)
    def _work():
        pltpu.sync_copy(eids_hbm.at[pl.ds(wid * PPW, PPW)], eidv)

        def vbody(v, cnts):
            ev = eidv[pl.ds(v * 16, 16)]
            rk = jnp.zeros((16,), jnp.int32)
            new = []
            for e in range(E):
                m = ev == e
                mi = m.astype(jnp.int32)
                pref = plsc.cumsum(mi)
                tot = jnp.sum(mi)
                rk = jnp.where(m, cnts[e] + pref - 1, rk)
                new.append(cnts[e] + tot)
            rankv_m[pl.ds(v * 16, 16)] = rk
            return tuple(new)

        cnts = lax.fori_loop(0, VPW, vbody,
                             tuple(jnp.int32(0) for _ in range(E)))

        hv = jnp.zeros((16,), jnp.int32)
        for e in range(E):
            hv = jnp.where(iota == e, cnts[e], hv)
        histv[...] = hv
        pltpu.sync_copy(histv, shh.at[wid])
        plsc.subcore_barrier()

        pltpu.sync_copy(shh, allh)
        base = jnp.zeros((16,), jnp.int32)
        tot = jnp.zeros((16,), jnp.int32)
        for j in range(NW):
            hj = allh[j, :]
            base = base + jnp.where(j < wid, hj, 0)
            tot = tot + hj
        padded = ((tot + (TM - 1)) >> 9) << 9
        o_incl = plsc.cumsum(padded)
        o_excl = o_incl - padded
        startv[...] = o_excl + base

        def pbody(v, _):
            ev = eidv[pl.ds(v * 16, 16)]
            st = plsc.load_gather(startv, [ev])
            rankv_m[pl.ds(v * 16, 16)] = st + rankv_m[pl.ds(v * 16, 16)]
            return 0

        lax.fori_loop(0, VPW, pbody, 0)

        half = PPW // 2
        for u in range(VPW // 2):
            idx0 = u * 32 + 2 * iota
            outv[pl.ds(u * 16, 16)] = plsc.load_gather(rankv_m, [idx0])
            outv[pl.ds(half + u * 16, 16)] = plsc.load_gather(rankv_m, [idx0 + 1])
        pltpu.sync_copy(outv.at[pl.ds(0, half)],
                        pos0_hbm.at[pl.ds(wid * half, half)])
        pltpu.sync_copy(outv.at[pl.ds(half, half)],
                        pos1_hbm.at[pl.ds(wid * half, half)])

        @pl.when(wid == 0)
        def _meta():
            tev = jnp.zeros((16,), jnp.int32)
            for e in range(E):
                oe = jnp.sum(jnp.where(iota == e, o_incl, 0))
                tev = tev + (oe <= iota * TM).astype(jnp.int32)
            last_e = jnp.max(jnp.where(tot > 0, iota, 0))
            tev = jnp.minimum(tev, last_e)
            nt = jnp.sum(padded) >> 9
            histv[...] = tev
            pltpu.sync_copy(histv, meta_hbm.at[pl.ds(0, 16)])
            histv[...] = jnp.zeros((16,), jnp.int32) + nt
            pltpu.sync_copy(histv, meta_hbm.at[pl.ds(16, 16)])


def _sc_compiler_params():
    cp = pltpu.CompilerParams()
    if "needs_layout_passes" in pltpu.CompilerParams.__dataclass_fields__:
        cp = dataclasses.replace(cp, needs_layout_passes=False)
    return cp


def _sortmeta(eids_flat):
    mesh = plsc.VectorSubcoreMesh(core_axis_name="c", subcore_axis_name="s")
    kern = functools.partial(
        pl.kernel,
        compiler_params=_sc_compiler_params(),
        out_type=(
            jax.ShapeDtypeStruct((T,), jnp.int32),
            jax.ShapeDtypeStruct((T,), jnp.int32),
            jax.ShapeDtypeStruct((32,), jnp.int32),
        ),
        mesh=mesh,
        scratch_types=[
            pltpu.VMEM((PPW,), jnp.int32),
            pltpu.VMEM((PPW,), jnp.int32),
            pltpu.VMEM((16,), jnp.int32),
            pltpu.VMEM((NW, 16), jnp.int32),
            pltpu.VMEM_SHARED((NW, 16), jnp.int32),
            pltpu.VMEM((16,), jnp.int32),
            pltpu.VMEM((PPW,), jnp.int32),
        ],
    )(_sortmeta_body)
    return kern(eids_flat)


def _dispatch_body(x_hbm, pos0_hbm, pos1_hbm, xs_hbm, rows, i0, i1, sem):
    cid = lax.axis_index("c")
    sid = lax.axis_index("s")
    w = sid * 2 + cid
    tpw = T // 32
    for chunk in range(2):
        base = w * tpw + chunk * 32
        pltpu.sync_copy(x_hbm.at[pl.ds(base, 32)], rows)
        pltpu.sync_copy(pos0_hbm.at[pl.ds(base, 32)], i0)
        pltpu.sync_copy(pos1_hbm.at[pl.ds(base, 32)], i1)
        pltpu.async_copy(rows, xs_hbm.at[i0], sem).wait()
        pltpu.async_copy(rows, xs_hbm.at[i1], sem).wait()


def _dispatch(x_flat, pos0, pos1):
    mesh = plsc.VectorSubcoreMesh(core_axis_name="c", subcore_axis_name="s")
    kern = functools.partial(
        pl.kernel,
        out_type=jax.ShapeDtypeStruct((ROWS, C), jnp.float32),
        mesh=mesh,
        scratch_types=[
            pltpu.VMEM((32, C), jnp.float32),
            pltpu.VMEM((32,), jnp.int32),
            pltpu.VMEM((32,), jnp.int32),
            pltpu.SemaphoreType.DMA,
        ],
    )(_dispatch_body)
    return kern(x_flat, pos0, pos1)


def _ffn_body(meta_ref, xs_ref, wfc_ref, bfc_ref, wproj_ref, bproj_ref, o_ref):
    j = pl.program_id(0)
    f = pl.program_id(1)
    nt = meta_ref[16]

    @pl.when(j < nt)
    def _active():
        h = jnp.dot(xs_ref[...], wfc_ref[0], preferred_element_type=jnp.float32)
        h = h + bfc_ref[0]
        h = 0.5 * h * (1.0 + jnp.tanh(0.7978845608028654
                                      * (h + 0.044715 * h * h * h)))
        yb = jnp.dot(h, wproj_ref[0], preferred_element_type=jnp.float32)

        @pl.when(f == 0)
        def _():
            o_ref[...] = yb + bproj_ref[0]

        @pl.when(f != 0)
        def _():
            o_ref[...] = o_ref[...] + yb

    @pl.when(jnp.logical_and(j >= nt, f == 0))
    def _inactive():
        o_ref[...] = jnp.zeros_like(o_ref)


def _ffn(meta, xs, Wfc, bfc, Wproj, bproj):
    def jpin(j, f, m):
        return jnp.minimum(j, m[16] - 1)

    def fpin(j, f, m):
        return jnp.where(j < m[16], f, GRID_F - 1)

    grid_spec = pltpu.PrefetchScalarGridSpec(
        num_scalar_prefetch=1,
        grid=(MAX_TILES, GRID_F),
        in_specs=[
            pl.BlockSpec((TM, C), lambda j, f, m: (jpin(j, f, m), 0)),
            pl.BlockSpec((1, C, TF), lambda j, f, m: (m[j], 0, fpin(j, f, m))),
            pl.BlockSpec((1, 1, TF), lambda j, f, m: (m[j], 0, fpin(j, f, m))),
            pl.BlockSpec((1, TF, C), lambda j, f, m: (m[j], fpin(j, f, m), 0)),
            pl.BlockSpec((1, 1, C), lambda j, f, m: (m[j], 0, 0)),
        ],
        out_specs=pl.BlockSpec((TM, C), lambda j, f, m: (j, 0)),
    )
    return pl.pallas_call(
        _ffn_body,
        grid_spec=grid_spec,
        out_shape=jax.ShapeDtypeStruct((ROWS, C), jnp.float32),
        compiler_params=pltpu.CompilerParams(
            dimension_semantics=("parallel", "arbitrary")),
    )(meta, xs, Wfc, bfc.reshape(E, 1, DFF), Wproj, bproj.reshape(E, 1, C))


def _gather_body(y_hbm, pos0_hbm, pos1_hbm, yc0_hbm, yc1_hbm, rows, idx, sem):
    cid = lax.axis_index("c")
    sid = lax.axis_index("s")
    w = sid * 2 + cid
    tpw = T // 32
    for chunk in range(2):
        base = w * tpw + chunk * 32
        pltpu.sync_copy(pos0_hbm.at[pl.ds(base, 32)], idx)
        pltpu.async_copy(y_hbm.at[idx], rows, sem).wait()
        pltpu.sync_copy(rows, yc0_hbm.at[pl.ds(base, 32)])
        pltpu.sync_copy(pos1_hbm.at[pl.ds(base, 32)], idx)
        pltpu.async_copy(y_hbm.at[idx], rows, sem).wait()
        pltpu.sync_copy(rows, yc1_hbm.at[pl.ds(base, 32)])


def _gather(y, pos0, pos1):
    mesh = plsc.VectorSubcoreMesh(core_axis_name="c", subcore_axis_name="s")
    kern = functools.partial(
        pl.kernel,
        out_type=(
            jax.ShapeDtypeStruct((T, C), jnp.float32),
            jax.ShapeDtypeStruct((T, C), jnp.float32),
        ),
        mesh=mesh,
        scratch_types=[
            pltpu.VMEM((32, C), jnp.float32),
            pltpu.VMEM((32,), jnp.int32),
            pltpu.SemaphoreType.DMA,
        ],
    )(_gather_body)
    return kern(y, pos0, pos1)


def _combine_body(y0_ref, y1_ref, g_ref, o_ref):
    g = g_ref[...]
    o_ref[...] = y0_ref[...] * g[:, 0:1] + y1_ref[...] * g[:, 1:2]


def _combine(yc0, yc1, gates):
    grid_m = T // TM
    return pl.pallas_call(
        _combine_body,
        grid=(grid_m,),
        out_shape=jax.ShapeDtypeStruct((T, C), jnp.float32),
        in_specs=[
            pl.BlockSpec((TM, C), lambda m: (m, 0)),
            pl.BlockSpec((TM, C), lambda m: (m, 0)),
            pl.BlockSpec((TM, 2), lambda m: (m, 0)),
        ],
        out_specs=pl.BlockSpec((TM, C), lambda m: (m, 0)),
    )(yc0, yc1, gates)


def kernel(x, Wg, bg, Wfc, bfc, Wproj, bproj):
    Bb, Tt, Cc = x.shape
    x_flat = x.reshape(Tt, Cc)

    eids, gates = _router(x_flat, Wg, bg)
    pos0, pos1, meta = _sortmeta(eids.reshape(PAIRS))
    xs = _dispatch(x_flat, pos0, pos1)
    y = _ffn(meta, xs, Wfc, bfc, Wproj, bproj)
    yc0, yc1 = _gather(y, pos0, pos1)
    out = _combine(yc0, yc1, gates)
    return out.reshape(Bb, Tt, Cc)


# speedup vs baseline: 2.7973x; 1.0004x over previous
import dataclasses
import functools

import jax
import jax.numpy as jnp
from jax import lax
from jax.experimental import pallas as pl
from jax.experimental.pallas import tpu as pltpu
from jax.experimental.pallas import tpu_sc as plsc

T = 2048
C = 2048
DFF = 8192
E = 8
TM = 512
TF = 1024
GRID_F = DFF // TF
MAX_TILES = 16
ROWS = MAX_TILES * TM
NW = 16
PAIRS = 2 * T
PPW = PAIRS // NW
VPW = PPW // 16


def _router_body(x_ref, wg_ref, bg_ref, eids_ref, gates_ref):
    s = jnp.dot(x_ref[...], wg_ref[...], preferred_element_type=jnp.float32)
    s = s + bg_ref[...]
    lane = lax.broadcasted_iota(jnp.int32, s.shape, 1)
    m1 = jnp.max(s, axis=1, keepdims=True)
    e1 = jnp.min(jnp.where(s == m1, lane, E), axis=1, keepdims=True)
    s2 = jnp.where(lane == e1, -jnp.inf, s)
    m2 = jnp.max(s2, axis=1, keepdims=True)
    e2 = jnp.min(jnp.where(s2 == m2, lane, E), axis=1, keepdims=True)
    g2 = 1.0 / (1.0 + jnp.exp(m1 - m2))
    g1 = 1.0 - g2
    eids_ref[...] = jnp.concatenate([e1, e2], axis=1).astype(jnp.int32)
    gates_ref[...] = jnp.concatenate([g1, g2], axis=1)


def _router(x_flat, Wg, bg):
    return pl.pallas_call(
        _router_body,
        out_shape=(
            jax.ShapeDtypeStruct((T, 2), jnp.int32),
            jax.ShapeDtypeStruct((T, 2), jnp.float32),
        ),
        in_specs=[
            pl.BlockSpec((T, C), lambda: (0, 0)),
            pl.BlockSpec((C, E), lambda: (0, 0)),
            pl.BlockSpec((1, E), lambda: (0, 0)),
        ],
        out_specs=(
            pl.BlockSpec((T, 2), lambda: (0, 0)),
            pl.BlockSpec((T, 2), lambda: (0, 0)),
        ),
    )(x_flat, Wg, bg.reshape(1, E))


def _sortmeta_body(eids_hbm, pos0_hbm, pos1_hbm, meta_hbm,
                   eidv, rankv_m, histv, allh, shh, startv, outv):
    cid = lax.axis_index("c")
    wid = lax.axis_index("s")
    iota = lax.iota(jnp.int32, 16)

    @pl.when(cid == 0)
    def _work():
        pltpu.sync_copy(eids_hbm.at[pl.ds(wid * PPW, PPW)], eidv)

        def vbody(v, cnts):
            ev = eidv[pl.ds(v * 16, 16)]
            rk = jnp.zeros((16,), jnp.int32)
            new = []
            for e in range(E):
                m = ev == e
                mi = m.astype(jnp.int32)
                pref = plsc.cumsum(mi)
                tot = jnp.sum(mi)
                rk = jnp.where(m, cnts[e] + pref - 1, rk)
                new.append(cnts[e] + tot)
            rankv_m[pl.ds(v * 16, 16)] = rk
            return tuple(new)

        cnts = lax.fori_loop(0, VPW, vbody,
                             tuple(jnp.int32(0) for _ in range(E)))

        hv = jnp.zeros((16,), jnp.int32)
        for e in range(E):
            hv = jnp.where(iota == e, cnts[e], hv)
        histv[...] = hv
        pltpu.sync_copy(histv, shh.at[wid])
        plsc.subcore_barrier()

        pltpu.sync_copy(shh, allh)
        base = jnp.zeros((16,), jnp.int32)
        tot = jnp.zeros((16,), jnp.int32)
        for j in range(NW):
            hj = allh[j, :]
            base = base + jnp.where(j < wid, hj, 0)
            tot = tot + hj
        padded = ((tot + (TM - 1)) >> 9) << 9
        o_incl = plsc.cumsum(padded)
        o_excl = o_incl - padded
        startv[...] = o_excl + base

        def pbody(v, _):
            ev = eidv[pl.ds(v * 16, 16)]
            st = plsc.load_gather(startv, [ev])
            rankv_m[pl.ds(v * 16, 16)] = st + rankv_m[pl.ds(v * 16, 16)]
            return 0

        lax.fori_loop(0, VPW, pbody, 0)

        half = PPW // 2
        for u in range(VPW // 2):
            idx0 = u * 32 + 2 * iota
            outv[pl.ds(u * 16, 16)] = plsc.load_gather(rankv_m, [idx0])
            outv[pl.ds(half + u * 16, 16)] = plsc.load_gather(rankv_m, [idx0 + 1])
        pltpu.sync_copy(outv.at[pl.ds(0, half)],
                        pos0_hbm.at[pl.ds(wid * half, half)])
        pltpu.sync_copy(outv.at[pl.ds(half, half)],
                        pos1_hbm.at[pl.ds(wid * half, half)])

        @pl.when(wid == 0)
        def _meta():
            tev = jnp.zeros((16,), jnp.int32)
            for e in range(E):
                oe = jnp.sum(jnp.where(iota == e, o_incl, 0))
                tev = tev + (oe <= iota * TM).astype(jnp.int32)
            last_e = jnp.max(jnp.where(tot > 0, iota, 0))
            tev = jnp.minimum(tev, last_e)
            nt = jnp.sum(padded) >> 9
            histv[...] = tev
            pltpu.sync_copy(histv, meta_hbm.at[pl.ds(0, 16)])
            histv[...] = jnp.zeros((16,), jnp.int32) + nt
            pltpu.sync_copy(histv, meta_hbm.at[pl.ds(16, 16)])


def _sc_compiler_params():
    cp = pltpu.CompilerParams()
    if "needs_layout_passes" in pltpu.CompilerParams.__dataclass_fields__:
        cp = dataclasses.replace(cp, needs_layout_passes=False)
    return cp


def _sortmeta(eids_flat):
    mesh = plsc.VectorSubcoreMesh(core_axis_name="c", subcore_axis_name="s")
    kern = functools.partial(
        pl.kernel,
        compiler_params=_sc_compiler_params(),
        out_type=(
            jax.ShapeDtypeStruct((T,), jnp.int32),
            jax.ShapeDtypeStruct((T,), jnp.int32),
            jax.ShapeDtypeStruct((32,), jnp.int32),
        ),
        mesh=mesh,
        scratch_types=[
            pltpu.VMEM((PPW,), jnp.int32),
            pltpu.VMEM((PPW,), jnp.int32),
            pltpu.VMEM((16,), jnp.int32),
            pltpu.VMEM((NW, 16), jnp.int32),
            pltpu.VMEM_SHARED((NW, 16), jnp.int32),
            pltpu.VMEM((16,), jnp.int32),
            pltpu.VMEM((PPW,), jnp.int32),
        ],
    )(_sortmeta_body)
    return kern(eids_flat)


def _dispatch_body(x_hbm, pos0_hbm, pos1_hbm, xs_hbm, rows, i0, i1, sem):
    cid = lax.axis_index("c")
    sid = lax.axis_index("s")
    w = sid * 2 + cid
    tpw = T // 32
    for chunk in range(2):
        base = w * tpw + chunk * 32
        pltpu.sync_copy(x_hbm.at[pl.ds(base, 32)], rows)
        pltpu.sync_copy(pos0_hbm.at[pl.ds(base, 32)], i0)
        pltpu.sync_copy(pos1_hbm.at[pl.ds(base, 32)], i1)
        pltpu.async_copy(rows, xs_hbm.at[i0], sem).wait()
        pltpu.async_copy(rows, xs_hbm.at[i1], sem).wait()


def _dispatch(x_flat, pos0, pos1):
    mesh = plsc.VectorSubcoreMesh(core_axis_name="c", subcore_axis_name="s")
    kern = functools.partial(
        pl.kernel,
        out_type=jax.ShapeDtypeStruct((ROWS, C), jnp.float32),
        mesh=mesh,
        scratch_types=[
            pltpu.VMEM((32, C), jnp.float32),
            pltpu.VMEM((32,), jnp.int32),
            pltpu.VMEM((32,), jnp.int32),
            pltpu.SemaphoreType.DMA,
        ],
    )(_dispatch_body)
    return kern(x_flat, pos0, pos1)


def _ffn_body(meta_ref, xs_ref, wfc_ref, bfc_ref, wproj_ref, bproj_ref, o_ref):
    j = pl.program_id(0)
    f = pl.program_id(1)
    nt = meta_ref[16]

    @pl.when(j < nt)
    def _active():
        h = jnp.dot(xs_ref[...], wfc_ref[0], preferred_element_type=jnp.float32)
        h = h + bfc_ref[0]
        h = 0.5 * h * (1.0 + jnp.tanh(0.7978845608028654
                                      * (h + 0.044715 * h * h * h)))
        yb = jnp.dot(h, wproj_ref[0], preferred_element_type=jnp.float32)

        @pl.when(f == 0)
        def _():
            o_ref[...] = yb + bproj_ref[0]

        @pl.when(f != 0)
        def _():
            o_ref[...] = o_ref[...] + yb

    @pl.when(jnp.logical_and(j >= nt, f == 0))
    def _inactive():
        o_ref[...] = jnp.zeros_like(o_ref)


def _ffn(meta, xs, Wfc, bfc, Wproj, bproj):
    def jpin(j, f, m):
        return jnp.minimum(j, m[16] - 1)

    def fpin(j, f, m):
        return jnp.where(j < m[16], f, GRID_F - 1)

    grid_spec = pltpu.PrefetchScalarGridSpec(
        num_scalar_prefetch=1,
        grid=(MAX_TILES, GRID_F),
        in_specs=[
            pl.BlockSpec((TM, C), lambda j, f, m: (jpin(j, f, m), 0)),
            pl.BlockSpec((1, C, TF), lambda j, f, m: (m[j], 0, fpin(j, f, m))),
            pl.BlockSpec((1, 1, TF), lambda j, f, m: (m[j], 0, fpin(j, f, m))),
            pl.BlockSpec((1, TF, C), lambda j, f, m: (m[j], fpin(j, f, m), 0)),
            pl.BlockSpec((1, 1, C), lambda j, f, m: (m[j], 0, 0)),
        ],
        out_specs=pl.BlockSpec((TM, C), lambda j, f, m: (j, 0)),
    )
    return pl.pallas_call(
        _ffn_body,
        grid_spec=grid_spec,
        out_shape=jax.ShapeDtypeStruct((ROWS, C), jnp.float32),
        compiler_params=pltpu.CompilerParams(
            dimension_semantics=("arbitrary", "arbitrary")),
    )(meta, xs, Wfc, bfc.reshape(E, 1, DFF), Wproj, bproj.reshape(E, 1, C))


def _gather_body(y_hbm, pos0_hbm, pos1_hbm, yc0_hbm, yc1_hbm, rows, idx, sem):
    cid = lax.axis_index("c")
    sid = lax.axis_index("s")
    w = sid * 2 + cid
    tpw = T // 32
    for chunk in range(2):
        base = w * tpw + chunk * 32
        pltpu.sync_copy(pos0_hbm.at[pl.ds(base, 32)], idx)
        pltpu.async_copy(y_hbm.at[idx], rows, sem).wait()
        pltpu.sync_copy(rows, yc0_hbm.at[pl.ds(base, 32)])
        pltpu.sync_copy(pos1_hbm.at[pl.ds(base, 32)], idx)
        pltpu.async_copy(y_hbm.at[idx], rows, sem).wait()
        pltpu.sync_copy(rows, yc1_hbm.at[pl.ds(base, 32)])


def _gather(y, pos0, pos1):
    mesh = plsc.VectorSubcoreMesh(core_axis_name="c", subcore_axis_name="s")
    kern = functools.partial(
        pl.kernel,
        out_type=(
            jax.ShapeDtypeStruct((T, C), jnp.float32),
            jax.ShapeDtypeStruct((T, C), jnp.float32),
        ),
        mesh=mesh,
        scratch_types=[
            pltpu.VMEM((32, C), jnp.float32),
            pltpu.VMEM((32,), jnp.int32),
            pltpu.SemaphoreType.DMA,
        ],
    )(_gather_body)
    return kern(y, pos0, pos1)


def _combine_body(y0_ref, y1_ref, g_ref, o_ref):
    g = g_ref[...]
    o_ref[...] = y0_ref[...] * g[:, 0:1] + y1_ref[...] * g[:, 1:2]


def _combine(yc0, yc1, gates):
    grid_m = T // TM
    return pl.pallas_call(
        _combine_body,
        grid=(grid_m,),
        out_shape=jax.ShapeDtypeStruct((T, C), jnp.float32),
        in_specs=[
            pl.BlockSpec((TM, C), lambda m: (m, 0)),
            pl.BlockSpec((TM, C), lambda m: (m, 0)),
            pl.BlockSpec((TM, 2), lambda m: (m, 0)),
        ],
        out_specs=pl.BlockSpec((TM, C), lambda m: (m, 0)),
    )(yc0, yc1, gates)


def kernel(x, Wg, bg, Wfc, bfc, Wproj, bproj):
    Bb, Tt, Cc = x.shape
    x_flat = x.reshape(Tt, Cc)

    eids, gates = _router(x_flat, Wg, bg)
    pos0, pos1, meta = _sortmeta(eids.reshape(PAIRS))
    xs = _dispatch(x_flat, pos0, pos1)
    y = _ffn(meta, xs, Wfc, bfc, Wproj, bproj)
    yc0, yc1 = _gather(y, pos0, pos1)
    out = _combine(yc0, yc1, gates)
    return out.reshape(Bb, Tt, Cc)
